# baseline (speedup 1.0000x reference)
import functools

import jax
import jax.numpy as jnp
from jax import lax
from jax.experimental import pallas as pl
from jax.experimental.pallas import tpu as pltpu
from jax.experimental.pallas import tpu_sc as plsc

NN = 100000
NB = 2048
HID = 128
REL = 825
RELP = 832
LAB = 800
NBINS = NB * RELP
HALF = NBINS // 2
SEGH = NB // 2
STRIPE = HALF // 16
ZB = 4096
CHUNK = 6256
TAIL = NN - 15 * CHUNK
HREG = 6272
ROWS = 98
NVH = HREG // 16


def _sc_hist(seg_h, idx_h, seg_t, idx_t):
    mesh = plsc.VectorSubcoreMesh(core_axis_name="c", subcore_axis_name="s")

    @functools.partial(
        pl.kernel,
        out_type=jax.ShapeDtypeStruct((NBINS,), jnp.float32),
        mesh=mesh,
        scratch_types=[
            pltpu.VMEM((ROWS * 128,), jnp.int32),
            pltpu.VMEM((ROWS * 128,), jnp.int32),
            pltpu.VMEM((ROWS * 128,), jnp.int32),
            pltpu.VMEM((ROWS * 128,), jnp.float32),
            pltpu.VMEM((ZB,), jnp.float32),
            pltpu.VMEM_SHARED((HALF + 8,), jnp.float32),
            pltpu.SemaphoreType.DMA,
            pltpu.SemaphoreType.DMA,
        ],
    )
    def hist_kernel(segh_hbm, idxh_hbm, segt_hbm, idxt_hbm, out,
                    segv, idxv, comb, vals, zbuf, acc, ld_sem, z_sem):
        half = lax.axis_index("c")
        pos = lax.axis_index("s")
        zero16f = jnp.zeros((16,), jnp.float32)
        neg16i = jnp.full((16,), -1, jnp.int32)

        @pl.when(pos < 15)
        def _():
            segv[pl.ds(CHUNK, 16)] = neg16i
            segv[pl.ds(HREG + CHUNK, 16)] = neg16i

        @pl.when(pos == 15)
        def _():
            for t in range(7):
                segv[pl.ds(TAIL + t * 16, 16)] = neg16i
                segv[pl.ds(HREG + TAIL + t * 16, 16)] = neg16i

        def fire_loads(n):
            pltpu.async_copy(segh_hbm.at[pl.ds(pos * CHUNK, n)],
                             segv.at[pl.ds(0, n)], ld_sem)
            pltpu.async_copy(idxh_hbm.at[pl.ds(pos * CHUNK, n)],
                             idxv.at[pl.ds(0, n)], ld_sem)
            pltpu.async_copy(segt_hbm.at[pl.ds(pos * CHUNK, n)],
                             segv.at[pl.ds(HREG, n)], ld_sem)
            pltpu.async_copy(idxt_hbm.at[pl.ds(pos * CHUNK, n)],
                             idxv.at[pl.ds(HREG, n)], ld_sem)

        def drain_loads(n):
            pltpu.make_async_copy(segh_hbm.at[pl.ds(pos * CHUNK, n)],
                                  segv.at[pl.ds(0, n)], ld_sem).wait()
            pltpu.make_async_copy(idxh_hbm.at[pl.ds(pos * CHUNK, n)],
                                  idxv.at[pl.ds(0, n)], ld_sem).wait()
            pltpu.make_async_copy(segt_hbm.at[pl.ds(pos * CHUNK, n)],
                                  segv.at[pl.ds(HREG, n)], ld_sem).wait()
            pltpu.make_async_copy(idxt_hbm.at[pl.ds(pos * CHUNK, n)],
                                  idxv.at[pl.ds(HREG, n)], ld_sem).wait()

        @pl.when(pos < 15)
        def _():
            fire_loads(CHUNK)

        @pl.when(pos == 15)
        def _():
            fire_loads(TAIL)

        def fill_zbuf(i, carry):
            zbuf[pl.ds(i * 16, 16)] = zero16f
            return carry

        lax.fori_loop(0, ZB // 16, fill_zbuf, 0)

        for i in range(STRIPE // ZB):
            pltpu.async_copy(zbuf, acc.at[pl.ds(pos * STRIPE + i * ZB, ZB)],
                             z_sem)

        @pl.when(pos == 15)
        def _():
            pltpu.async_copy(zbuf.at[pl.ds(0, 8)], acc.at[pl.ds(HALF, 8)],
                             z_sem)

        @pl.when(pos < 15)
        def _():
            drain_loads(CHUNK)

        @pl.when(pos == 15)
        def _():
            drain_loads(TAIL)

        lo = half * SEGH

        def make_side(j0, j1, sgn):
            def body(j, carry):
                sv = segv[pl.ds(j * 16, 16)]
                iv = idxv[pl.ds(j * 16, 16)]
                ok = (sv >= lo) & (sv < lo + SEGH)
                comb[pl.ds(j * 16, 16)] = jnp.where(
                    ok, (sv - lo) * RELP + iv, HALF)
                vals[pl.ds(j * 16, 16)] = jnp.where(ok, sgn, 0.0)
                return carry

            lax.fori_loop(j0, j1, body, 0)

        make_side(0, NVH, 1.0)
        make_side(NVH, 2 * NVH, -1.0)

        for i in range(STRIPE // ZB):
            pltpu.make_async_copy(
                zbuf, acc.at[pl.ds(pos * STRIPE + i * ZB, ZB)], z_sem).wait()

        @pl.when(pos == 15)
        def _():
            pltpu.make_async_copy(zbuf.at[pl.ds(0, 8)],
                                  acc.at[pl.ds(HALF, 8)], z_sem).wait()

        plsc.subcore_barrier()

        pltpu.sync_copy(vals, acc.at[comb], add=True)

        plsc.subcore_barrier()

        pltpu.sync_copy(acc.at[pl.ds(pos * STRIPE, STRIPE)],
                        out.at[pl.ds(half * HALF + pos * STRIPE, STRIPE)])

    return hist_kernel(seg_h, idx_h, seg_t, idx_t)


def _tc_head(hist, rel, w, b2, labels3):
    BR = 256
    grid = NB // BR

    def body(h_ref, r_ref, w_ref, b_ref, lab_ref, logits_ref, loss_ref):
        i = pl.program_id(0)
        e = jnp.dot(h_ref[...], r_ref[...], preferred_element_type=jnp.float32)
        logits = jnp.dot(e, w_ref[...], preferred_element_type=jnp.float32) + b_ref[...]
        logits_ref[...] = logits
        m = jnp.max(logits, axis=-1, keepdims=True)
        lse = jnp.log(jnp.sum(jnp.exp(logits - m), axis=-1, keepdims=True)) + m
        lab = lab_ref[0, 0, :]
        cols = lax.broadcasted_iota(jnp.int32, (BR, LAB), 1)
        picked = jnp.sum(jnp.where(cols == lab[:, None], logits, 0.0),
                         axis=-1, keepdims=True)
        part = (jnp.sum(lse - picked) * (1.0 / NB)).reshape(1, 1)

        @pl.when(i == 0)
        def _():
            loss_ref[...] = jnp.zeros((1, 1), jnp.float32)

        loss_ref[...] += part

    return pl.pallas_call(
        body,
        grid=(grid,),
        in_specs=[
            pl.BlockSpec((BR, RELP), lambda i: (i, 0)),
            pl.BlockSpec((RELP, HID), lambda i: (0, 0)),
            pl.BlockSpec((HID, LAB), lambda i: (0, 0)),
            pl.BlockSpec((1, LAB), lambda i: (0, 0)),
            pl.BlockSpec((1, 1, BR), lambda i: (i, 0, 0)),
        ],
        out_specs=[
            pl.BlockSpec((BR, LAB), lambda i: (i, 0)),
            pl.BlockSpec((1, 1), lambda i: (0, 0)),
        ],
        out_shape=[
            jax.ShapeDtypeStruct((NB, LAB), jnp.float32),
            jax.ShapeDtypeStruct((1, 1), jnp.float32),
        ],
    )(hist, rel, w, b2, labels3)


def kernel(node_idx_h, edge_idx_h, seg_ids_h, node_idx_t, edge_idx_t,
           seg_ids_t, labels, rel_table, pat_table, W, b):
    del edge_idx_h, edge_idx_t, pat_table
    hist = _sc_hist(seg_ids_h.astype(jnp.int32), node_idx_h.astype(jnp.int32),
                    seg_ids_t.astype(jnp.int32), node_idx_t.astype(jnp.int32))
    hist = hist.reshape(NB, RELP)
    rel_pad = jnp.zeros((RELP, HID), jnp.float32).at[:REL].set(rel_table)
    labels3 = labels.astype(jnp.int32).reshape(NB // 256, 1, 256)
    logits, loss = _tc_head(hist, rel_pad, W, b.reshape(1, LAB), labels3)
    return logits, loss[0, 0]
